# baseline (speedup 1.0000x reference)
import jax
import jax.numpy as jnp
from jax.experimental import pallas as pl
from jax.experimental.pallas import tpu as pltpu

_BN = 4096


def _outer(x_ref, p_ref, o_hbm):
    i = pl.program_id(0)
    x = x_ref[...]

    def _inner(idx, o_ref):
        tt = idx[0]
        acc = jax.lax.dot_general(
            x, p_ref[tt],
            dimension_numbers=(((1,), (1,)), ((), ())),
            preferred_element_type=jnp.float32,
        )
        o_ref[0] = jnp.where(acc < 0, jnp.float32(0.0), jnp.float32(1.0))

    pipe = pltpu.emit_pipeline(
        _inner,
        grid=(o_hbm.shape[0],),
        out_specs=[pl.BlockSpec((1, _BN, o_hbm.shape[2]), lambda t: (t, i, 0))],
        _explicit_indices=True,
    )
    pipe(o_hbm)


def kernel(input_points, planes):
    n, d = input_points.shape
    t, h, _ = planes.shape
    return pl.pallas_call(
        _outer,
        grid=(n // _BN,),
        in_specs=[
            pl.BlockSpec((_BN, d), lambda i: (i, 0)),
            pl.BlockSpec((t, h, d), lambda i: (0, 0, 0)),
        ],
        out_specs=pl.BlockSpec(memory_space=pl.ANY),
        out_shape=jax.ShapeDtypeStruct((t, n, h), jnp.float32),
    )(input_points, planes)
